# baseline (speedup 1.0000x reference)
import jax
import jax.numpy as jnp
from jax.experimental import pallas as pl
from jax.experimental.pallas import tpu as pltpu

ROWS = 8192
COLS = 4096
BLOCK_ROWS = 512
CHUNK = 128
NCHUNK = COLS // CHUNK


def _cumsum_sublanes(x):
    n = x.shape[0]
    d = 1
    while d < n:
        shifted = jnp.concatenate(
            [jnp.zeros((d,) + x.shape[1:], x.dtype), x[:-d]], axis=0)
        x = x + shifted
        d *= 2
    return x


def _scan_block(x_ref, m_ref, e_ref, o_ref, carry_ref):
    i = pl.program_id(0)
    blk = x_ref[...]
    blk16 = blk.astype(jnp.bfloat16)

    chunk_excl = jax.lax.dot(blk16, e_ref[...],
                             preferred_element_type=jnp.float32)
    ce16 = chunk_excl.astype(jnp.bfloat16)

    row_tot = jnp.sum(blk, axis=1, keepdims=True)
    row_incl = _cumsum_sublanes(row_tot)
    row_excl = row_incl - row_tot

    carry = jnp.where(i == 0, 0.0, carry_ref[0:1, 0:1])
    base = row_excl + carry

    for k in range(NCHUNK):
        sl = slice(k * CHUNK, (k + 1) * CHUNK)
        lhs = jnp.concatenate([blk16[:, sl], ce16], axis=1)
        o_ref[:, sl] = jax.lax.dot(
            lhs, m_ref[:, sl], preferred_element_type=jnp.float32
        ) + base

    carry_ref[0:1, 0:1] = carry + row_incl[BLOCK_ROWS - 1:BLOCK_ROWS, :]


@jax.jit
def kernel(x):
    grid = ROWS // BLOCK_ROWS

    ii = jnp.arange(CHUNK)
    t = (ii[:, None] <= ii[None, :]).astype(jnp.bfloat16)
    kk = jnp.arange(NCHUNK)
    cc = jnp.arange(COLS)
    sel = (kk[:, None] == (cc[None, :] // CHUNK)).astype(jnp.bfloat16)
    m = jnp.concatenate(
        [jnp.tile(t, (1, NCHUNK)), sel], axis=0)
    e = (cc[:, None] < kk[None, :] * CHUNK).astype(jnp.bfloat16)

    return pl.pallas_call(
        _scan_block,
        grid=(grid,),
        in_specs=[
            pl.BlockSpec((BLOCK_ROWS, COLS), lambda i: (i, 0)),
            pl.BlockSpec((CHUNK + NCHUNK, COLS), lambda i: (0, 0)),
            pl.BlockSpec((COLS, NCHUNK), lambda i: (0, 0)),
        ],
        out_specs=pl.BlockSpec((BLOCK_ROWS, COLS), lambda i: (i, 0)),
        out_shape=jax.ShapeDtypeStruct((ROWS, COLS), jnp.float32),
        scratch_shapes=[pltpu.VMEM((1, 1), jnp.float32)],
        compiler_params=pltpu.CompilerParams(
            dimension_semantics=("arbitrary",),
        ),
    )(x, m, e)


# speedup vs baseline: 1.0100x; 1.0100x over previous
import jax
import jax.numpy as jnp
from jax.experimental import pallas as pl
from jax.experimental.pallas import tpu as pltpu

ROWS = 8192
COLS = 4096
BLOCK_ROWS = 512
CHUNK = 128
NCHUNK = COLS // CHUNK


def _cumsum_sublanes(x):
    n = x.shape[0]
    d = 1
    while d < n:
        shifted = jnp.concatenate(
            [jnp.zeros((d,) + x.shape[1:], x.dtype), x[:-d]], axis=0)
        x = x + shifted
        d *= 2
    return x


def _scan_block(x_ref, t_ref, e_ref, p_ref, o_ref, carry_ref):
    i = pl.program_id(0)
    blk = x_ref[...]
    t = t_ref[...]
    blk16 = blk.astype(jnp.bfloat16)

    chunk_excl = jax.lax.dot(blk16, e_ref[...],
                             preferred_element_type=jnp.float32)

    row_tot = jnp.sum(blk, axis=1, keepdims=True)
    row_incl = _cumsum_sublanes(row_tot)
    row_excl = row_incl - row_tot

    carry = jnp.where(i == 0, 0.0, carry_ref[0:1, 0:1])

    base = row_excl + carry
    offs = jax.lax.dot(chunk_excl, p_ref[...],
                       preferred_element_type=jnp.float32)

    for k in range(NCHUNK):
        sl = slice(k * CHUNK, (k + 1) * CHUNK)
        o_ref[:, sl] = jax.lax.dot(
            blk16[:, sl], t, preferred_element_type=jnp.float32
        ) + (offs[:, sl] + base)

    carry_ref[0:1, 0:1] = carry + row_incl[BLOCK_ROWS - 1:BLOCK_ROWS, :]


@jax.jit
def kernel(x):
    grid = ROWS // BLOCK_ROWS

    ii = jnp.arange(CHUNK)
    t = (ii[:, None] <= ii[None, :]).astype(jnp.bfloat16)
    cc = jnp.arange(COLS)
    kk = jnp.arange(NCHUNK)
    e = (cc[:, None] < kk[None, :] * CHUNK).astype(jnp.bfloat16)
    p = (kk[:, None] == (cc[None, :] // CHUNK)).astype(jnp.float32)

    return pl.pallas_call(
        _scan_block,
        grid=(grid,),
        in_specs=[
            pl.BlockSpec((BLOCK_ROWS, COLS), lambda i: (i, 0)),
            pl.BlockSpec((CHUNK, CHUNK), lambda i: (0, 0)),
            pl.BlockSpec((COLS, NCHUNK), lambda i: (0, 0)),
            pl.BlockSpec((NCHUNK, COLS), lambda i: (0, 0)),
        ],
        out_specs=pl.BlockSpec((BLOCK_ROWS, COLS), lambda i: (i, 0)),
        out_shape=jax.ShapeDtypeStruct((ROWS, COLS), jnp.float32),
        scratch_shapes=[pltpu.VMEM((1, 1), jnp.float32)],
        compiler_params=pltpu.CompilerParams(
            dimension_semantics=("arbitrary",),
        ),
    )(x, t, e, p)


# speedup vs baseline: 1.0725x; 1.0619x over previous
import jax
import jax.numpy as jnp
from jax.experimental import pallas as pl
from jax.experimental.pallas import tpu as pltpu

ROWS = 8192
COLS = 4096
BLOCK_ROWS = 256
CHUNK = 128
NCHUNK = COLS // CHUNK


def _cumsum_sublanes(x):
    n = x.shape[0]
    d = 1
    while d < n:
        shifted = jnp.concatenate(
            [jnp.zeros((d,) + x.shape[1:], x.dtype), x[:-d]], axis=0)
        x = x + shifted
        d *= 2
    return x


def _scan_block(x_ref, t_ref, e_ref, p_ref, o_ref, carry_ref):
    i = pl.program_id(0)
    blk = x_ref[...]
    t = t_ref[...]
    blk16 = blk.astype(jnp.bfloat16)

    chunk_excl = jax.lax.dot(blk16, e_ref[...],
                             preferred_element_type=jnp.float32)

    row_tot = jnp.sum(blk, axis=1, keepdims=True)
    row_incl = _cumsum_sublanes(row_tot)
    row_excl = row_incl - row_tot

    carry = jnp.where(i == 0, 0.0, carry_ref[0:1, 0:1])

    base = row_excl + carry
    offs = jax.lax.dot(chunk_excl, p_ref[...],
                       preferred_element_type=jnp.float32)

    for k in range(NCHUNK):
        sl = slice(k * CHUNK, (k + 1) * CHUNK)
        o_ref[:, sl] = jax.lax.dot(
            blk16[:, sl], t, preferred_element_type=jnp.float32
        ) + (offs[:, sl] + base)

    carry_ref[0:1, 0:1] = carry + row_incl[BLOCK_ROWS - 1:BLOCK_ROWS, :]


@jax.jit
def kernel(x):
    grid = ROWS // BLOCK_ROWS

    ii = jnp.arange(CHUNK)
    t = (ii[:, None] <= ii[None, :]).astype(jnp.bfloat16)
    cc = jnp.arange(COLS)
    kk = jnp.arange(NCHUNK)
    e = (cc[:, None] < kk[None, :] * CHUNK).astype(jnp.bfloat16)
    p = (kk[:, None] == (cc[None, :] // CHUNK)).astype(jnp.float32)

    return pl.pallas_call(
        _scan_block,
        grid=(grid,),
        in_specs=[
            pl.BlockSpec((BLOCK_ROWS, COLS), lambda i: (i, 0)),
            pl.BlockSpec((CHUNK, CHUNK), lambda i: (0, 0)),
            pl.BlockSpec((COLS, NCHUNK), lambda i: (0, 0)),
            pl.BlockSpec((NCHUNK, COLS), lambda i: (0, 0)),
        ],
        out_specs=pl.BlockSpec((BLOCK_ROWS, COLS), lambda i: (i, 0)),
        out_shape=jax.ShapeDtypeStruct((ROWS, COLS), jnp.float32),
        scratch_shapes=[pltpu.VMEM((1, 1), jnp.float32)],
        compiler_params=pltpu.CompilerParams(
            dimension_semantics=("arbitrary",),
        ),
    )(x, t, e, p)
